# Initial kernel scaffold; baseline (speedup 1.0000x reference)
import jax
import jax.numpy as jnp
from jax.experimental import pallas as pl


def kernel(x, edge_index, W1, b1, gn1_weight, gn1_bias, gn1_mean_scale, W2, b2, gn2_weight, gn2_bias, gn2_mean_scale):
    raise NotImplementedError("write your pallas kernel here")


# speedup vs baseline: 7.4475x; 7.4475x over previous
import functools

import jax
import jax.numpy as jnp
from jax import lax
from jax.experimental import pallas as pl
from jax.experimental.pallas import tpu as pltpu
from jax.experimental.pallas import tpu_sc as plsc

N = 10000
D = 128
E = 320000
EPS = 1e-5

NC = 2
NS = 16
NW = NC * NS

CHUNK = 128
CH = 80
EPT = CH * CHUNK
E_PAD = NW * EPT
NPAD = 10240
RPS = NPAD // NS
ZB = RPS // CHUNK

_mesh = plsc.VectorSubcoreMesh(core_axis_name="c", subcore_axis_name="s")


@functools.partial(
    pl.kernel,
    out_type=jax.ShapeDtypeStruct((NC * NPAD, 16), jnp.float32),
    mesh=_mesh,
    scratch_types=[
        pltpu.VMEM((CHUNK,), jnp.int32),
        pltpu.VMEM((CHUNK, 16), jnp.float32),
        pltpu.VMEM_SHARED((NPAD, 16), jnp.float32),
    ],
)
def _deg_kernel(cols_hbm, const_hbm, out_hbm, cidx_v, ones_v, acc):
    cid = lax.axis_index("c")
    sid = lax.axis_index("s")
    wid = cid * NS + sid
    base = sid * RPS
    for k in range(ZB):
        pltpu.sync_copy(const_hbm.at[pl.ds(CHUNK, CHUNK)],
                        acc.at[pl.ds(base + k * CHUNK, CHUNK)])
    pltpu.sync_copy(const_hbm.at[pl.ds(0, CHUNK)], ones_v)
    plsc.subcore_barrier()

    def body(j, carry):
        ebase = pl.multiple_of(wid * EPT + j * CHUNK, CHUNK)
        pltpu.sync_copy(cols_hbm.at[pl.ds(ebase, CHUNK)], cidx_v)
        pltpu.sync_copy(ones_v, acc.at[cidx_v], add=True)
        return carry

    lax.fori_loop(0, CH, body, 0)
    plsc.subcore_barrier()
    for k in range(ZB):
        rb = base + k * CHUNK
        pltpu.sync_copy(acc.at[pl.ds(rb, CHUNK)],
                        out_hbm.at[pl.ds(cid * NPAD + rb, CHUNK)])


@functools.partial(
    pl.kernel,
    out_type=jax.ShapeDtypeStruct((NC * NPAD, D), jnp.float32),
    mesh=_mesh,
    scratch_types=[
        pltpu.VMEM((CHUNK,), jnp.int32),
        pltpu.VMEM((CHUNK,), jnp.int32),
        pltpu.VMEM((CHUNK, D), jnp.float32),
        pltpu.SemaphoreType.DMA,
        pltpu.VMEM_SHARED((NPAD, D), jnp.float32),
    ],
)
def _spmm_kernel(h_hbm, rows_hbm, cols_hbm, out_hbm,
                 ridx_v, cidx_v, buf_v, sem, acc):
    cid = lax.axis_index("c")
    sid = lax.axis_index("s")
    wid = cid * NS + sid
    base = sid * RPS
    for k in range(ZB):
        pltpu.sync_copy(h_hbm.at[pl.ds(N + 112, CHUNK)],
                        acc.at[pl.ds(base + k * CHUNK, CHUNK)])
    plsc.subcore_barrier()

    def body(j, carry):
        ebase = pl.multiple_of(wid * EPT + j * CHUNK, CHUNK)
        pltpu.sync_copy(rows_hbm.at[pl.ds(ebase, CHUNK)], ridx_v)
        pltpu.sync_copy(cols_hbm.at[pl.ds(ebase, CHUNK)], cidx_v)
        pltpu.async_copy(h_hbm.at[ridx_v], buf_v, sem).wait()
        pltpu.sync_copy(buf_v, acc.at[cidx_v], add=True)
        return carry

    lax.fori_loop(0, CH, body, 0)
    plsc.subcore_barrier()
    for k in range(ZB):
        rb = base + k * CHUNK
        pltpu.sync_copy(acc.at[pl.ds(rb, CHUNK)],
                        out_hbm.at[pl.ds(cid * NPAD + rb, CHUNK)])


def _dis_from_degp(degp):
    d0 = degp[pl.ds(0, N), :]
    d1 = degp[pl.ds(NPAD, N), :]
    deg = d0 + d1 + 1.0
    return lax.rsqrt(deg)[:, 0:1]


def _tc1_body(x_ref, w1_ref, degp_ref, out_ref):
    dis = _dis_from_degp(degp_ref)
    h = jnp.dot(x_ref[...], w1_ref[...], preferred_element_type=jnp.float32)
    out_ref[pl.ds(0, N), :] = h * dis
    out_ref[pl.ds(N, NPAD - N), :] = jnp.zeros((NPAD - N, D), jnp.float32)


def _graph_norm_relu(t, w, b, ms):
    mean = jnp.sum(t, axis=0, keepdims=True) * (1.0 / N)
    c = t - mean * ms
    var = jnp.sum(c * c, axis=0, keepdims=True) * (1.0 / N)
    return jnp.maximum(w * c * lax.rsqrt(var + EPS) + b, 0.0)


def _tc_mid_body(s_ref, hp_ref, degp_ref, gnw_ref, gnb_ref, gnms_ref,
                 b1_ref, w2_ref, out_ref):
    dis = _dis_from_degp(degp_ref)
    hp = hp_ref[pl.ds(0, N), :]
    t = (s_ref[pl.ds(0, N), :] + s_ref[pl.ds(NPAD, N), :] + hp) * dis
    t = t + b1_ref[...]
    g = _graph_norm_relu(t, gnw_ref[...], gnb_ref[...], gnms_ref[...])
    h2 = jnp.dot(g, w2_ref[...], preferred_element_type=jnp.float32)
    out_ref[pl.ds(0, N), :] = h2 * dis
    out_ref[pl.ds(N, NPAD - N), :] = jnp.zeros((NPAD - N, D), jnp.float32)


def _tc_final_body(s_ref, hp_ref, degp_ref, gnw_ref, gnb_ref, gnms_ref,
                   b2_ref, out_ref):
    dis = _dis_from_degp(degp_ref)
    hp = hp_ref[pl.ds(0, N), :]
    t = (s_ref[pl.ds(0, N), :] + s_ref[pl.ds(NPAD, N), :] + hp) * dis
    t = t + b2_ref[...]
    out_ref[...] = _graph_norm_relu(t, gnw_ref[...], gnb_ref[...],
                                    gnms_ref[...])


_tc1 = pl.pallas_call(
    _tc1_body, out_shape=jax.ShapeDtypeStruct((NPAD, D), jnp.float32))
_tc_mid = pl.pallas_call(
    _tc_mid_body, out_shape=jax.ShapeDtypeStruct((NPAD, D), jnp.float32))
_tc_final = pl.pallas_call(
    _tc_final_body, out_shape=jax.ShapeDtypeStruct((N, D), jnp.float32))


def kernel(x, edge_index, W1, b1, gn1_weight, gn1_bias, gn1_mean_scale,
           W2, b2, gn2_weight, gn2_bias, gn2_mean_scale):
    pad = jnp.full((E_PAD - E,), N, jnp.int32)
    rows_p = jnp.concatenate([edge_index[0], pad])
    cols_p = jnp.concatenate([edge_index[1], pad])
    const = jnp.concatenate([jnp.ones((CHUNK, 16), jnp.float32),
                             jnp.zeros((CHUNK, 16), jnp.float32)])

    degp = _deg_kernel(cols_p, const)
    hp1 = _tc1(x, W1, degp)
    s1 = _spmm_kernel(hp1, rows_p, cols_p)
    hp2 = _tc_mid(s1, hp1, degp, gn1_weight.reshape(1, D),
                  gn1_bias.reshape(1, D), gn1_mean_scale.reshape(1, D),
                  b1.reshape(1, D), W2)
    s2 = _spmm_kernel(hp2, rows_p, cols_p)
    out = _tc_final(s2, hp2, degp, gn2_weight.reshape(1, D),
                    gn2_bias.reshape(1, D), gn2_mean_scale.reshape(1, D),
                    b2.reshape(1, D))
    return out


# speedup vs baseline: 9.5736x; 1.2855x over previous
import functools

import jax
import jax.numpy as jnp
from jax import lax
from jax.experimental import pallas as pl
from jax.experimental.pallas import tpu as pltpu
from jax.experimental.pallas import tpu_sc as plsc

N = 10000
D = 128
E = 320000
EPS = 1e-5

NC = 2
NS = 16
NW = NC * NS

CHUNK = 128
CH = 80
EPT = CH * CHUNK
E_PAD = NW * EPT
NPAD = 10240
RPS = NPAD // NS
ZB = RPS // CHUNK
NBUF = 2
HALF = CH // 2
NGRP = HALF // NBUF
NGRP_DEG = CH // NBUF

_mesh = plsc.VectorSubcoreMesh(core_axis_name="c", subcore_axis_name="s")


@functools.partial(
    pl.kernel,
    out_type=jax.ShapeDtypeStruct((NC * NPAD, 16), jnp.float32),
    mesh=_mesh,
    scratch_types=[
        pltpu.VMEM((CH, CHUNK), jnp.int32),
        pltpu.VMEM((CHUNK, 16), jnp.float32),
        pltpu.VMEM_SHARED((NPAD, 16), jnp.float32),
    ] + [pltpu.SemaphoreType.DMA] * NBUF,
)
def _deg_kernel(cols_hbm, const_hbm, out_hbm, cidx_v, ones_v, acc, *sems):
    cid = lax.axis_index("c")
    sid = lax.axis_index("s")
    wid = cid * NS + sid
    base = sid * RPS
    for k in range(ZB):
        pltpu.sync_copy(const_hbm.at[pl.ds(CHUNK, CHUNK)],
                        acc.at[pl.ds(base + k * CHUNK, CHUNK)])
    pltpu.sync_copy(const_hbm.at[pl.ds(0, CHUNK)], ones_v)
    pltpu.sync_copy(cols_hbm.at[pl.ds(wid * CH, CH)], cidx_v)
    plsc.subcore_barrier()

    def fire(j, b):
        pltpu.async_copy(ones_v, acc.at[cidx_v.at[j]], sems[b], add=True)

    def drain(j, b):
        pltpu.make_async_copy(ones_v, acc.at[cidx_v.at[j]], sems[b]).wait()

    for b in range(NBUF):
        fire(b, b)

    def group(g, carry):
        for b in range(NBUF):
            j = g * NBUF + b
            drain(j, b)
            fire(j + NBUF, b)
        return carry

    lax.fori_loop(0, NGRP_DEG - 1, group, 0)
    for b in range(NBUF):
        drain((NGRP_DEG - 1) * NBUF + b, b)
    plsc.subcore_barrier()
    for k in range(ZB):
        rb = base + k * CHUNK
        pltpu.sync_copy(acc.at[pl.ds(rb, CHUNK)],
                        out_hbm.at[pl.ds(cid * NPAD + rb, CHUNK)])


@functools.partial(
    pl.kernel,
    out_type=jax.ShapeDtypeStruct((NC * NPAD, D), jnp.float32),
    mesh=_mesh,
    scratch_types=[
        pltpu.VMEM((HALF, 2, CHUNK), jnp.int32),
        pltpu.VMEM_SHARED((NPAD, D), jnp.float32),
    ] + [pltpu.VMEM((CHUNK, D), jnp.float32)] * NBUF
      + [pltpu.SemaphoreType.DMA] * (2 * NBUF),
)
def _spmm_kernel(h_hbm, eidx_hbm, out_hbm, idx_v, acc, *bufs_sems):
    bufs = bufs_sems[:NBUF]
    gsem = bufs_sems[NBUF:2 * NBUF]
    ssem = bufs_sems[2 * NBUF:]
    cid = lax.axis_index("c")
    sid = lax.axis_index("s")
    wid = cid * NS + sid
    base = sid * RPS
    for k in range(ZB):
        pltpu.sync_copy(h_hbm.at[pl.ds(N + 112, CHUNK)],
                        acc.at[pl.ds(base + k * CHUNK, CHUNK)])

    def fire_gather(j, b):
        pltpu.async_copy(h_hbm.at[idx_v.at[j, 0]], bufs[b], gsem[b])

    def wait_gather(j, b):
        pltpu.make_async_copy(h_hbm.at[idx_v.at[j, 0]], bufs[b],
                              gsem[b]).wait()

    def fire_scatter(j, b):
        pltpu.async_copy(bufs[b], acc.at[idx_v.at[j, 1]], ssem[b], add=True)

    def wait_scatter(j, b):
        pltpu.make_async_copy(bufs[b], acc.at[idx_v.at[j, 1]],
                              ssem[b]).wait()

    for half in range(2):
        pltpu.sync_copy(eidx_hbm.at[pl.ds(wid * CH + half * HALF, HALF)],
                        idx_v)
        if half == 0:
            plsc.subcore_barrier()

        for b in range(NBUF):
            fire_gather(b, b)

        def group(g, carry):
            for b in range(NBUF):
                j = g * NBUF + b
                wait_gather(j, b)
                fire_scatter(j, b)
            for b in range(NBUF):
                j = g * NBUF + b
                wait_scatter(j, b)
                fire_gather(j + NBUF, b)
            return carry

        lax.fori_loop(0, NGRP - 1, group, 0)
        for b in range(NBUF):
            j = (NGRP - 1) * NBUF + b
            wait_gather(j, b)
            fire_scatter(j, b)
        for b in range(NBUF):
            wait_scatter((NGRP - 1) * NBUF + b, b)
    plsc.subcore_barrier()
    for k in range(ZB):
        rb = base + k * CHUNK
        pltpu.sync_copy(acc.at[pl.ds(rb, CHUNK)],
                        out_hbm.at[pl.ds(cid * NPAD + rb, CHUNK)])


def _dis_from_degp(degp):
    d0 = degp[pl.ds(0, N), :]
    d1 = degp[pl.ds(NPAD, N), :]
    deg = d0 + d1 + 1.0
    return lax.rsqrt(deg)[:, 0:1]


def _tc1_body(x_ref, w1_ref, degp_ref, out_ref):
    dis = _dis_from_degp(degp_ref)
    h = jnp.dot(x_ref[...], w1_ref[...], preferred_element_type=jnp.float32)
    out_ref[pl.ds(0, N), :] = h * dis
    out_ref[pl.ds(N, NPAD - N), :] = jnp.zeros((NPAD - N, D), jnp.float32)


def _graph_norm_relu(t, w, b, ms):
    mean = jnp.sum(t, axis=0, keepdims=True) * (1.0 / N)
    c = t - mean * ms
    var = jnp.sum(c * c, axis=0, keepdims=True) * (1.0 / N)
    return jnp.maximum(w * c * lax.rsqrt(var + EPS) + b, 0.0)


def _tc_mid_body(s_ref, hp_ref, degp_ref, gnw_ref, gnb_ref, gnms_ref,
                 b1_ref, w2_ref, out_ref):
    dis = _dis_from_degp(degp_ref)
    hp = hp_ref[pl.ds(0, N), :]
    t = (s_ref[pl.ds(0, N), :] + s_ref[pl.ds(NPAD, N), :] + hp) * dis
    t = t + b1_ref[...]
    g = _graph_norm_relu(t, gnw_ref[...], gnb_ref[...], gnms_ref[...])
    h2 = jnp.dot(g, w2_ref[...], preferred_element_type=jnp.float32)
    out_ref[pl.ds(0, N), :] = h2 * dis
    out_ref[pl.ds(N, NPAD - N), :] = jnp.zeros((NPAD - N, D), jnp.float32)


def _tc_final_body(s_ref, hp_ref, degp_ref, gnw_ref, gnb_ref, gnms_ref,
                   b2_ref, out_ref):
    dis = _dis_from_degp(degp_ref)
    hp = hp_ref[pl.ds(0, N), :]
    t = (s_ref[pl.ds(0, N), :] + s_ref[pl.ds(NPAD, N), :] + hp) * dis
    t = t + b2_ref[...]
    out_ref[...] = _graph_norm_relu(t, gnw_ref[...], gnb_ref[...],
                                    gnms_ref[...])


_tc1 = pl.pallas_call(
    _tc1_body, out_shape=jax.ShapeDtypeStruct((NPAD, D), jnp.float32))
_tc_mid = pl.pallas_call(
    _tc_mid_body, out_shape=jax.ShapeDtypeStruct((NPAD, D), jnp.float32))
_tc_final = pl.pallas_call(
    _tc_final_body, out_shape=jax.ShapeDtypeStruct((N, D), jnp.float32))


def kernel(x, edge_index, W1, b1, gn1_weight, gn1_bias, gn1_mean_scale,
           W2, b2, gn2_weight, gn2_bias, gn2_mean_scale):
    pad = jnp.full((E_PAD - E,), N, jnp.int32)
    rows_p = jnp.concatenate([edge_index[0], pad]).reshape(E_PAD // CHUNK,
                                                           CHUNK)
    cols_p = jnp.concatenate([edge_index[1], pad]).reshape(E_PAD // CHUNK,
                                                           CHUNK)
    eidx = jnp.stack([rows_p, cols_p], axis=1)
    const = jnp.concatenate([jnp.ones((CHUNK, 16), jnp.float32),
                             jnp.zeros((CHUNK, 16), jnp.float32)])

    degp = _deg_kernel(cols_p, const)
    hp1 = _tc1(x, W1, degp)
    s1 = _spmm_kernel(hp1, eidx)
    hp2 = _tc_mid(s1, hp1, degp, gn1_weight.reshape(1, D),
                  gn1_bias.reshape(1, D), gn1_mean_scale.reshape(1, D),
                  b1.reshape(1, D), W2)
    s2 = _spmm_kernel(hp2, eidx)
    out = _tc_final(s2, hp2, degp, gn2_weight.reshape(1, D),
                    gn2_bias.reshape(1, D), gn2_mean_scale.reshape(1, D),
                    b2.reshape(1, D))
    return out


# speedup vs baseline: 24.4560x; 2.5545x over previous
import functools

import jax
import jax.numpy as jnp
from jax import lax
from jax.experimental import pallas as pl
from jax.experimental.pallas import tpu as pltpu
from jax.experimental.pallas import tpu_sc as plsc

N = 10000
D = 128
E = 320000
EPS = 1e-5

NC = 2
NS = 16
NW = NC * NS

CHUNK = 128
CH = 80
EPT = CH * CHUNK
E_PAD = NW * EPT
NPAD = 10240
RPS = NPAD // NS
ZB = RPS // CHUNK
NBUF = 2
HALF = CH // 2
NGRP = HALF // NBUF
NGRP_DEG = CH // NBUF

_mesh = plsc.VectorSubcoreMesh(core_axis_name="c", subcore_axis_name="s")


@functools.partial(
    pl.kernel,
    out_type=jax.ShapeDtypeStruct((NC * NPAD, 16), jnp.float32),
    mesh=_mesh,
    scratch_types=[
        pltpu.VMEM((CH, CHUNK), jnp.int32),
        pltpu.VMEM((CHUNK, 16), jnp.float32),
        pltpu.VMEM_SHARED((NPAD, 16), jnp.float32),
    ] + [pltpu.SemaphoreType.DMA] * NBUF,
)
def _deg_kernel(cols_hbm, const_hbm, out_hbm, cidx_v, ones_v, acc, *sems):
    cid = lax.axis_index("c")
    sid = lax.axis_index("s")
    wid = cid * NS + sid
    base = sid * RPS
    for k in range(ZB):
        pltpu.sync_copy(const_hbm.at[pl.ds(CHUNK, CHUNK)],
                        acc.at[pl.ds(base + k * CHUNK, CHUNK)])
    pltpu.sync_copy(const_hbm.at[pl.ds(0, CHUNK)], ones_v)
    pltpu.sync_copy(cols_hbm.at[pl.ds(wid * CH, CH)], cidx_v)
    plsc.subcore_barrier()

    def fire(j, b):
        pltpu.async_copy(ones_v, acc.at[cidx_v.at[j]], sems[b], add=True)

    def drain(j, b):
        pltpu.make_async_copy(ones_v, acc.at[cidx_v.at[j]], sems[b]).wait()

    for b in range(NBUF):
        fire(b, b)

    def group(g, carry):
        for b in range(NBUF):
            j = g * NBUF + b
            drain(j, b)
            fire(j + NBUF, b)
        return carry

    lax.fori_loop(0, NGRP_DEG - 1, group, 0)
    for b in range(NBUF):
        drain((NGRP_DEG - 1) * NBUF + b, b)
    plsc.subcore_barrier()
    for k in range(ZB):
        rb = base + k * CHUNK
        pltpu.sync_copy(acc.at[pl.ds(rb, CHUNK)],
                        out_hbm.at[pl.ds(cid * NPAD + rb, CHUNK)])


@functools.partial(
    pl.kernel,
    out_type=jax.ShapeDtypeStruct((NC * NPAD, D), jnp.float32),
    mesh=_mesh,
    scratch_types=[
        pltpu.VMEM((HALF, 2, CHUNK), jnp.int32),
        pltpu.VMEM_SHARED((NPAD, D), jnp.float32),
    ] + [pltpu.VMEM((CHUNK, D), jnp.float32)] * NBUF
      + [pltpu.SemaphoreType.DMA] * (2 * NBUF),
)
def _spmm_kernel(h_hbm, eidx_hbm, out_hbm, idx_v, acc, *bufs_sems):
    bufs = bufs_sems[:NBUF]
    gsem = bufs_sems[NBUF:2 * NBUF]
    ssem = bufs_sems[2 * NBUF:]
    cid = lax.axis_index("c")
    sid = lax.axis_index("s")
    wid = cid * NS + sid
    base = sid * RPS
    for k in range(ZB):
        pltpu.sync_copy(h_hbm.at[pl.ds(N + 112, CHUNK)],
                        acc.at[pl.ds(base + k * CHUNK, CHUNK)])

    def fire_gather(j, b):
        pltpu.async_copy(h_hbm.at[idx_v.at[j, 0]], bufs[b], gsem[b])

    def wait_gather(j, b):
        pltpu.make_async_copy(h_hbm.at[idx_v.at[j, 0]], bufs[b],
                              gsem[b]).wait()

    def fire_scatter(j, b):
        pltpu.async_copy(bufs[b], acc.at[idx_v.at[j, 1]], ssem[b], add=True)

    def wait_scatter(j, b):
        pltpu.make_async_copy(bufs[b], acc.at[idx_v.at[j, 1]],
                              ssem[b]).wait()

    for half in range(2):
        pltpu.sync_copy(eidx_hbm.at[pl.ds(wid * CH + half * HALF, HALF)],
                        idx_v)
        if half == 0:
            plsc.subcore_barrier()

        for b in range(NBUF):
            fire_gather(b, b)

        def group(g, carry):
            for b in range(NBUF):
                j = g * NBUF + b
                wait_gather(j, b)
                fire_scatter(j, b)
            for b in range(NBUF):
                j = g * NBUF + b
                wait_scatter(j, b)
                fire_gather(j + NBUF, b)
            return carry

        lax.fori_loop(0, NGRP - 1, group, 0)
        for b in range(NBUF):
            j = (NGRP - 1) * NBUF + b
            wait_gather(j, b)
            fire_scatter(j, b)
        for b in range(NBUF):
            wait_scatter((NGRP - 1) * NBUF + b, b)
    plsc.subcore_barrier()
    for k in range(ZB):
        rb = base + k * CHUNK
        pltpu.sync_copy(acc.at[pl.ds(rb, CHUNK)],
                        out_hbm.at[pl.ds(cid * NPAD + rb, CHUNK)])


def _dis_from_degp(degp):
    d0 = degp[pl.ds(0, N), :]
    d1 = degp[pl.ds(NPAD, N), :]
    deg = d0 + d1 + 1.0
    return lax.rsqrt(deg)[:, 0:1]


def _tc1_body(x_ref, w1_ref, degp_ref, out_ref):
    dis = _dis_from_degp(degp_ref)
    h = jnp.dot(x_ref[...], w1_ref[...], preferred_element_type=jnp.float32)
    out_ref[pl.ds(0, N), :] = h * dis
    out_ref[pl.ds(N, NPAD - N), :] = jnp.zeros((NPAD - N, D), jnp.float32)


def _graph_norm_relu(t, w, b, ms):
    mean = jnp.sum(t, axis=0, keepdims=True) * (1.0 / N)
    c = t - mean * ms
    var = jnp.sum(c * c, axis=0, keepdims=True) * (1.0 / N)
    return jnp.maximum(w * c * lax.rsqrt(var + EPS) + b, 0.0)


def _tc_mid_body(s_ref, hp_ref, degp_ref, gnw_ref, gnb_ref, gnms_ref,
                 b1_ref, w2_ref, out_ref):
    dis = _dis_from_degp(degp_ref)
    hp = hp_ref[pl.ds(0, N), :]
    t = (s_ref[pl.ds(0, N), :] + s_ref[pl.ds(NPAD, N), :] + hp) * dis
    t = t + b1_ref[...]
    g = _graph_norm_relu(t, gnw_ref[...], gnb_ref[...], gnms_ref[...])
    h2 = jnp.dot(g, w2_ref[...], preferred_element_type=jnp.float32)
    out_ref[pl.ds(0, N), :] = h2 * dis
    out_ref[pl.ds(N, NPAD - N), :] = jnp.zeros((NPAD - N, D), jnp.float32)


def _tc_final_body(s_ref, hp_ref, degp_ref, gnw_ref, gnb_ref, gnms_ref,
                   b2_ref, out_ref):
    dis = _dis_from_degp(degp_ref)
    hp = hp_ref[pl.ds(0, N), :]
    t = (s_ref[pl.ds(0, N), :] + s_ref[pl.ds(NPAD, N), :] + hp) * dis
    t = t + b2_ref[...]
    out_ref[...] = _graph_norm_relu(t, gnw_ref[...], gnb_ref[...],
                                    gnms_ref[...])


_tc1 = pl.pallas_call(
    _tc1_body, out_shape=jax.ShapeDtypeStruct((NPAD, D), jnp.float32))
_tc_mid = pl.pallas_call(
    _tc_mid_body, out_shape=jax.ShapeDtypeStruct((NPAD, D), jnp.float32))
_tc_final = pl.pallas_call(
    _tc_final_body, out_shape=jax.ShapeDtypeStruct((N, D), jnp.float32))


def kernel(x, edge_index, W1, b1, gn1_weight, gn1_bias, gn1_mean_scale,
           W2, b2, gn2_weight, gn2_bias, gn2_mean_scale):
    pad = N + jnp.arange(E_PAD - E, dtype=jnp.int32) % (NPAD - N)
    rows_p = jnp.concatenate([edge_index[0], pad]).reshape(E_PAD // CHUNK,
                                                           CHUNK)
    cols_p = jnp.concatenate([edge_index[1], pad]).reshape(E_PAD // CHUNK,
                                                           CHUNK)
    eidx = jnp.stack([rows_p, cols_p], axis=1)
    const = jnp.concatenate([jnp.ones((CHUNK, 16), jnp.float32),
                             jnp.zeros((CHUNK, 16), jnp.float32)])

    degp = _deg_kernel(cols_p, const)
    hp1 = _tc1(x, W1, degp)
    s1 = _spmm_kernel(hp1, eidx)
    hp2 = _tc_mid(s1, hp1, degp, gn1_weight.reshape(1, D),
                  gn1_bias.reshape(1, D), gn1_mean_scale.reshape(1, D),
                  b1.reshape(1, D), W2)
    s2 = _spmm_kernel(hp2, eidx)
    out = _tc_final(s2, hp2, degp, gn2_weight.reshape(1, D),
                    gn2_bias.reshape(1, D), gn2_mean_scale.reshape(1, D),
                    b2.reshape(1, D))
    return out
